# baseline (speedup 1.0000x reference)
import functools

import jax
import jax.numpy as jnp
from jax import lax
from jax.experimental import pallas as pl
from jax.experimental.pallas import tpu as pltpu
from jax.experimental.pallas import tpu_sc as plsc

_CHUNK = 8


def kernel(r_idx, kv):
    b, p2, w2, c_kv = kv.shape
    topk = r_idx.shape[2]
    total = b * p2 * topk
    blk = w2 * c_kv

    nc, ns = 2, 16
    nw = nc * ns

    lo_n, hi_n = 96, 104
    n_lo = (nw * hi_n - total) // (hi_n - lo_n)
    lo_rows = n_lo * lo_n
    max_chunks = hi_n // _CHUNK
    lo_chunks = lo_n // _CHUNK

    kv_flat = kv.reshape(b * p2, blk)
    g_idx = (r_idx + (jnp.arange(b, dtype=r_idx.dtype) * p2)[:, None, None])
    g_idx = g_idx.reshape(total).astype(jnp.int32)

    mesh = plsc.VectorSubcoreMesh(
        core_axis_name="c", subcore_axis_name="s",
        num_cores=nc, num_subcores=ns,
    )

    @functools.partial(
        pl.kernel,
        out_type=jax.ShapeDtypeStruct((total, blk), kv.dtype),
        mesh=mesh,
        scratch_types=[
            pltpu.VMEM((hi_n,), jnp.int32),
            pltpu.VMEM((_CHUNK, blk), jnp.float32),
            pltpu.SemaphoreType.DMA,
        ],
    )
    def gather_rows(kv_hbm, idx_hbm, out_hbm, idx_v, buf, sem):
        w = lax.axis_index("s") * nc + lax.axis_index("c")
        is_lo = w < n_lo
        base = jnp.where(is_lo, lo_n * w, lo_rows + hi_n * (w - n_lo))
        base = pl.multiple_of(base, 8)

        @pl.when(is_lo)
        def _():
            pltpu.sync_copy(
                idx_hbm.at[pl.ds(base, lo_n)], idx_v.at[pl.ds(0, lo_n)]
            )

        @pl.when(jnp.logical_not(is_lo))
        def _():
            pltpu.sync_copy(idx_hbm.at[pl.ds(base, hi_n)], idx_v)

        for t in range(max_chunks):
            def chunk(t=t):
                pltpu.async_copy(
                    kv_hbm.at[idx_v.at[pl.ds(_CHUNK * t, _CHUNK)]],
                    buf,
                    sem,
                ).wait()
                pltpu.sync_copy(
                    buf, out_hbm.at[pl.ds(base + _CHUNK * t, _CHUNK)]
                )

            if t < lo_chunks:
                chunk()
            else:
                pl.when(jnp.logical_not(is_lo))(chunk)

    out = gather_rows(kv_flat, g_idx)
    return out.reshape(b, p2, topk, w2, c_kv)
